# baseline (speedup 1.0000x reference)
import functools

import jax
import jax.numpy as jnp
from jax import lax
from jax.experimental import pallas as pl
from jax.experimental.pallas import tpu as pltpu
from jax.experimental.pallas import tpu_sc as plsc

_BATCH = 4
_SEQ = 2048
_DIM = 128
_NC = 2
_NS = 16
_NW = _NC * _NS
_W = _SEQ // _NW
_NPAIR = _BATCH // 2
_SCALE = 11.313708498984761


def _sc_embed(x, table, pos_enc):
    mesh = plsc.VectorSubcoreMesh(core_axis_name="c", subcore_axis_name="s")

    @functools.partial(
        pl.kernel,
        mesh=mesh,
        out_type=jax.ShapeDtypeStruct((_BATCH, _SEQ, _DIM), jnp.float32),
        scratch_types=[
            pltpu.VMEM((_BATCH * _W,), jnp.int32),
            pltpu.VMEM((_NPAIR, 2 * _W, _DIM), jnp.float32),
            pltpu.VMEM((_W, _DIM), jnp.float32),
            pltpu.SemaphoreType.DMA,
            pltpu.SemaphoreType.DMA,
            pltpu.SemaphoreType.DMA,
            pltpu.SemaphoreType.DMA,
        ],
    )
    def embed_kernel(idx_hbm, table_hbm, pos_hbm, out_hbm,
                     idx_v, rows_v, pos_v, isem, psem, gsem, osem):
        wid = lax.axis_index("s") * _NC + lax.axis_index("c")
        s_off = wid * _W
        icps = [
            pltpu.async_copy(
                idx_hbm.at[b, pl.ds(s_off, _W)], idx_v.at[pl.ds(b * _W, _W)], isem
            )
            for b in range(_BATCH)
        ]
        pcp = pltpu.async_copy(pos_hbm.at[pl.ds(s_off, _W)], pos_v, psem)
        gcps = []
        for w in range(_NPAIR):
            icps[2 * w].wait()
            icps[2 * w + 1].wait()
            gcps.append(
                pltpu.async_copy(
                    table_hbm.at[idx_v.at[pl.ds(w * 2 * _W, 2 * _W)]],
                    rows_v.at[w],
                    gsem,
                )
            )
        pcp.wait()
        ocps = []
        half = _W // 2
        for w in range(_NPAIR):
            gcps[w].wait()
            for sub in range(2):

                @plsc.parallel_loop(sub * half, (sub + 1) * half, unroll=2)
                def _scale_add(r, w=w):
                    for j in range(_DIM // 16):
                        sl = pl.ds(j * 16, 16)
                        p = pos_v[r, sl]
                        rows_v[w, r, sl] = rows_v[w, r, sl] * _SCALE + p
                        rows_v[w, r + _W, sl] = rows_v[w, r + _W, sl] * _SCALE + p

                for h in range(2):
                    ocps.append(
                        pltpu.async_copy(
                            rows_v.at[w, pl.ds(h * _W + sub * half, half)],
                            out_hbm.at[2 * w + h, pl.ds(s_off + sub * half, half)],
                            osem,
                        )
                    )
        for cp in ocps:
            cp.wait()

    return embed_kernel(x, table, pos_enc)


def kernel(x, table, pos_enc):
    return _sc_embed(x.astype(jnp.int32), table, pos_enc)


# speedup vs baseline: 1.0150x; 1.0150x over previous
import functools

import jax
import jax.numpy as jnp
from jax import lax
from jax.experimental import pallas as pl
from jax.experimental.pallas import tpu as pltpu
from jax.experimental.pallas import tpu_sc as plsc

_BATCH = 4
_SEQ = 2048
_DIM = 128
_NC = 2
_NS = 16
_NW = _NC * _NS
_W = _SEQ // _NW
_NPAIR = _BATCH // 2
_SCALE = 11.313708498984761


def _sc_embed(x, table, pos_enc):
    mesh = plsc.VectorSubcoreMesh(core_axis_name="c", subcore_axis_name="s")

    @functools.partial(
        pl.kernel,
        mesh=mesh,
        out_type=jax.ShapeDtypeStruct((_BATCH, _SEQ, _DIM), jnp.float32),
        scratch_types=[
            pltpu.VMEM((_BATCH * _W,), jnp.int32),
            pltpu.VMEM((_NPAIR, 2 * _W, _DIM), jnp.float32),
            pltpu.VMEM((_W, _DIM), jnp.float32),
            pltpu.SemaphoreType.DMA,
            pltpu.SemaphoreType.DMA,
            pltpu.SemaphoreType.DMA,
            pltpu.SemaphoreType.DMA,
        ],
    )
    def embed_kernel(idx_hbm, table_hbm, pos_hbm, out_hbm,
                     idx_v, rows_v, pos_v, isem, psem, gsem, osem):
        wid = lax.axis_index("s") * _NC + lax.axis_index("c")
        s_off = wid * _W
        icps = [
            pltpu.async_copy(
                idx_hbm.at[b, pl.ds(s_off, _W)], idx_v.at[pl.ds(b * _W, _W)], isem
            )
            for b in range(_BATCH)
        ]
        gcps = []
        for w in range(_NPAIR):
            icps[2 * w].wait()
            icps[2 * w + 1].wait()
            gcps.append(
                pltpu.async_copy(
                    table_hbm.at[idx_v.at[pl.ds(w * 2 * _W, 2 * _W)]],
                    rows_v.at[w],
                    gsem,
                )
            )
        pcp = pltpu.async_copy(pos_hbm.at[pl.ds(s_off, _W)], pos_v, psem)
        pcp.wait()
        ocps = []
        for w in range(_NPAIR):
            gcps[w].wait()

            @plsc.parallel_loop(0, _W, unroll=2)
            def _scale_add(r, w=w):
                for j in range(_DIM // 16):
                    sl = pl.ds(j * 16, 16)
                    p = pos_v[r, sl]
                    rows_v[w, r, sl] = rows_v[w, r, sl] * _SCALE + p
                    rows_v[w, r + _W, sl] = rows_v[w, r + _W, sl] * _SCALE + p

            for h in range(2):
                ocps.append(
                    pltpu.async_copy(
                        rows_v.at[w, pl.ds(h * _W, _W)],
                        out_hbm.at[2 * w + h, pl.ds(s_off, _W)],
                        osem,
                    )
                )
        for cp in ocps:
            cp.wait()

    return embed_kernel(x, table, pos_enc)


def kernel(x, table, pos_enc):
    return _sc_embed(x.astype(jnp.int32), table, pos_enc)


# speedup vs baseline: 1.0416x; 1.0262x over previous
import functools

import jax
import jax.numpy as jnp
from jax import lax
from jax.experimental import pallas as pl
from jax.experimental.pallas import tpu as pltpu
from jax.experimental.pallas import tpu_sc as plsc

_BATCH = 4
_SEQ = 2048
_DIM = 128
_NC = 2
_NS = 16
_NW = _NC * _NS
_W = _SEQ // _NW
_NPAIR = _BATCH // 2
_SCALE = 11.313708498984761


def _sc_embed(x, table, pos_enc):
    mesh = plsc.VectorSubcoreMesh(core_axis_name="c", subcore_axis_name="s")

    @functools.partial(
        pl.kernel,
        mesh=mesh,
        out_type=jax.ShapeDtypeStruct((_BATCH, _SEQ, _DIM), jnp.float32),
        scratch_types=[
            pltpu.VMEM((_BATCH * _W,), jnp.int32),
            pltpu.VMEM((_NPAIR, 2 * _W, _DIM), jnp.float32),
            pltpu.VMEM((_W, _DIM), jnp.float32),
            pltpu.SemaphoreType.DMA,
            pltpu.SemaphoreType.DMA,
            pltpu.SemaphoreType.DMA,
            pltpu.SemaphoreType.DMA,
        ],
    )
    def embed_kernel(idx_hbm, table_hbm, pos_hbm, out_hbm,
                     idx_v, rows_v, pos_v, isem, psem, gsem, osem):
        wid = lax.axis_index("s") * _NC + lax.axis_index("c")
        s_off = wid * _W
        icps = [
            pltpu.async_copy(
                idx_hbm.at[b, pl.ds(s_off, _W)], idx_v.at[pl.ds(b * _W, _W)], isem
            )
            for b in range(_BATCH)
        ]
        pcp = pltpu.async_copy(pos_hbm.at[pl.ds(s_off, _W)], pos_v, psem)
        gcps = []
        for w in range(_NPAIR):
            icps[2 * w].wait()
            icps[2 * w + 1].wait()
            gcps.append(
                pltpu.async_copy(
                    table_hbm.at[idx_v.at[pl.ds(w * 2 * _W, 2 * _W)]],
                    rows_v.at[w],
                    gsem,
                )
            )
        pcp.wait()
        ocps = []
        for w in range(_NPAIR):
            gcps[w].wait()

            @plsc.parallel_loop(0, _W, unroll=1)
            def _scale_add(r, w=w):
                for j in range(_DIM // 16):
                    sl = pl.ds(j * 16, 16)
                    p = pos_v[r, sl]
                    rows_v[w, r, sl] = rows_v[w, r, sl] * _SCALE + p
                    rows_v[w, r + _W, sl] = rows_v[w, r + _W, sl] * _SCALE + p

            for h in range(2):
                ocps.append(
                    pltpu.async_copy(
                        rows_v.at[w, pl.ds(h * _W, _W)],
                        out_hbm.at[2 * w + h, pl.ds(s_off, _W)],
                        osem,
                    )
                )
        for cp in ocps:
            cp.wait()

    return embed_kernel(x, table, pos_enc)


def kernel(x, table, pos_enc):
    return _sc_embed(x.astype(jnp.int32), table, pos_enc)
